# baseline (speedup 1.0000x reference)
import functools

import jax
import jax.numpy as jnp
from jax import lax
from jax.experimental import pallas as pl
from jax.experimental.pallas import tpu as pltpu
from jax.experimental.pallas import tpu_sc as plsc

N_NODES = 10000
EMB = 128
NW = 32
RPW = 313
N_PAD = NW * RPW
CHUNK = 256
FEV = EMB // 16


def _spmm_sc(x_hbm, cols, vr, bounds):
    n_chunks = vr.shape[0]
    mesh = plsc.VectorSubcoreMesh(core_axis_name="c", subcore_axis_name="s",
                                  num_cores=2, num_subcores=16)

    @functools.partial(
        pl.kernel,
        mesh=mesh,
        out_type=jax.ShapeDtypeStruct((N_PAD * EMB,), jnp.float32),
        scratch_types=[
            pltpu.VMEM((48,), jnp.int32),
            [pltpu.VMEM((CHUNK,), jnp.int32) for _ in range(4)],
            [pltpu.VMEM((2, CHUNK), jnp.int32) for _ in range(4)],
            [pltpu.VMEM((CHUNK, EMB), jnp.float32) for _ in range(2)],
            pltpu.VMEM((RPW * EMB,), jnp.float32),
            [pltpu.SemaphoreType.DMA for _ in range(4)],
            [pltpu.SemaphoreType.DMA for _ in range(4)],
            [pltpu.SemaphoreType.DMA for _ in range(2)],
        ],
        compiler_params=pltpu.CompilerParams(needs_layout_passes=False),
    )
    def k(x_ref, cols_ref, vr_ref, bnd_ref, out_ref,
          bnd_v, cols_v, vr_v, gath_v, acc_v, csem, vsem, gsem):
        wid = lax.axis_index("s") * 2 + lax.axis_index("c")
        row_base = wid * RPW
        iota16 = lax.iota(jnp.int32, 16)
        z16 = jnp.zeros((16,), jnp.int32)
        o16 = jnp.full((16,), 1, jnp.int32)
        rbv = jnp.full((16,), row_base, jnp.int32)

        pltpu.sync_copy(bnd_ref, bnd_v)
        widv = jnp.full((16,), wid, dtype=jnp.int32)
        e_lo = plsc.load_gather(bnd_v, [widv])[0]
        e_hi = plsc.load_gather(bnd_v, [widv + 1])[0]

        c_lo = e_lo // CHUNK
        c_hi = lax.div(e_hi + (CHUNK - 1), CHUNK)
        nc4 = lax.div(c_hi - c_lo + 3, 4)

        def clamp(c):
            return lax.min(c, n_chunks - 1)

        def stage(c, i):
            pltpu.async_copy(cols_ref.at[pl.ds(clamp(c) * CHUNK, CHUNK)],
                             cols_v[i], csem[i])
            pltpu.async_copy(vr_ref.at[clamp(c)], vr_v[i], vsem[i])

        def wait_cols(i):
            pltpu.make_async_copy(cols_ref.at[pl.ds(0, CHUNK)],
                                  cols_v[i], csem[i]).wait()

        def wait_vr(i):
            pltpu.make_async_copy(vr_ref.at[0], vr_v[i], vsem[i]).wait()

        def wait_gath(i):
            pltpu.make_async_copy(x_ref.at[pl.ds(0, CHUNK)],
                                  gath_v[i], gsem[i]).wait()

        stage(c_lo, 0)
        stage(c_lo + 1, 1)

        zeros = jnp.zeros((16,), jnp.float32)

        def zero_body(t, _):
            acc_v[pl.ds(t * 16, 16)] = zeros
            return 0

        lax.fori_loop(0, RPW * FEV, zero_body, 0)

        wait_cols(0)
        pltpu.async_copy(x_ref.at[cols_v[0]], gath_v[0], gsem[0])

        def compute(c, vr_i, gath_i):
            base = c * CHUNK
            k0 = lax.max(e_lo - base, 0)
            k1 = lax.min(e_hi - base, CHUNK)
            k0v = jnp.full((16,), k0, jnp.int32)
            k1v = jnp.full((16,), k1, jnp.int32)

            def group_body(g, _):
                basev = jnp.full((16,), g * 16, jnp.int32)
                for i in range(16):
                    ev = basev + i
                    vv = plsc.bitcast(plsc.load_gather(vr_i, [z16, ev]),
                                      jnp.float32)
                    rv = plsc.load_gather(vr_i, [o16, ev])
                    okv = jnp.logical_and(ev >= k0v, ev < k1v)
                    vvm = jnp.where(okv, vv, 0.0)
                    rr = jnp.where(okv, (rv - rbv) * EMB, 0)
                    for j in range(FEV):
                        cj = iota16 + (j * 16)
                        gv = plsc.load_gather(gath_i, [ev, cj])
                        plsc.addupdate_scatter(acc_v, [rr + cj], gv * vvm)
                return 0

            lax.fori_loop(k0 // 16, lax.div(k1 + 15, 16), group_body, 0)

        def outer_body(it, _):
            c0 = c_lo + it * 4
            for b in range(4):
                c = c0 + b
                gb = b % 2
                stage(c + 2, (b + 2) % 4)
                wait_cols((b + 1) % 4)
                pltpu.async_copy(x_ref.at[cols_v[(b + 1) % 4]],
                                 gath_v[1 - gb], gsem[1 - gb])
                wait_gath(gb)
                wait_vr(b)
                compute(c, vr_v[b], gath_v[gb])
            return 0

        lax.fori_loop(0, nc4, outer_body, 0)

        wait_vr(0)
        wait_cols(1)
        wait_vr(1)
        wait_gath(0)

        pltpu.sync_copy(acc_v, out_ref.at[pl.ds(row_base * EMB, RPW * EMB)])

    out = k(x_hbm, cols, vr, bounds)
    return out.reshape(N_PAD, EMB)


def _mean_tc(x0, a1, a2, a3, b1, b2, b3):
    blk = N_PAD // 4

    def body(x0_r, a1_r, a2_r, a3_r, b1_r, b2_r, b3_r, o1_r, o2_r):
        x = x0_r[...]
        o1_r[...] = (x + a1_r[...] + a2_r[...] + a3_r[...]) * 0.25
        o2_r[...] = (x + b1_r[...] + b2_r[...] + b3_r[...]) * 0.25

    spec = pl.BlockSpec((blk, EMB), lambda i: (i, 0))
    o1, o2 = pl.pallas_call(
        body,
        grid=(4,),
        in_specs=[spec] * 7,
        out_specs=[spec, spec],
        out_shape=[jax.ShapeDtypeStruct((N_PAD, EMB), jnp.float32)] * 2,
    )(x0, a1, a2, a3, b1, b2, b3)
    return o1, o2


def _prep_mat(rc_v):
    rows, cols, vals = rc_v
    e = rows.shape[0]
    ep = ((e + CHUNK - 1) // CHUNK) * CHUNK
    pad = ep - e
    bounds = jnp.searchsorted(
        rows, jnp.arange(33, dtype=jnp.int32) * RPW).astype(jnp.int32)
    bounds = jnp.pad(bounds, (0, 15))
    rows_p = jnp.pad(rows.astype(jnp.int32), (0, pad),
                     constant_values=N_PAD - 1)
    cols_p = jnp.pad(cols.astype(jnp.int32), (0, pad))
    vals_p = jnp.pad(vals, (0, pad))
    nc = ep // CHUNK
    vr = jnp.stack([
        lax.bitcast_convert_type(vals_p, jnp.int32).reshape(nc, CHUNK),
        rows_p.reshape(nc, CHUNK),
    ], axis=1)
    return cols_p, vr, bounds


def kernel(ego_embeddings, sub1, sub2):
    x0 = jnp.pad(ego_embeddings, ((0, N_PAD - N_NODES), (0, 0)))

    def propagate(mats):
        outs = []
        x = x0
        for rc_v in mats:
            cols_p, vr, bounds = _prep_mat(rc_v)
            x = _spmm_sc(x, cols_p, vr, bounds)
            outs.append(x)
        return outs

    a1, a2, a3 = propagate(sub1)
    b1, b2, b3 = propagate(sub2)
    o1, o2 = _mean_tc(x0, a1, a2, a3, b1, b2, b3)
    o1 = o1[:N_NODES]
    o2 = o2[:N_NODES]
    half = N_NODES // 2
    return (o1[:half], o1[half:], o2[:half], o2[half:])


# speedup vs baseline: 2.2807x; 2.2807x over previous
import functools

import jax
import jax.numpy as jnp
from jax import lax
from jax.experimental import pallas as pl
from jax.experimental.pallas import tpu as pltpu
from jax.experimental.pallas import tpu_sc as plsc

N_NODES = 10000
EMB = 128
NW = 32
RPW = 313
N_PAD = NW * RPW
CHUNK = 256
FEV = EMB // 16


def _spmm_sc(x_hbm, cols, vr, bounds):
    n_chunks = vr.shape[0]
    mesh = plsc.VectorSubcoreMesh(core_axis_name="c", subcore_axis_name="s",
                                  num_cores=2, num_subcores=16)

    @functools.partial(
        pl.kernel,
        mesh=mesh,
        out_type=jax.ShapeDtypeStruct((N_PAD * EMB,), jnp.float32),
        scratch_types=[
            pltpu.VMEM((48,), jnp.int32),
            [pltpu.VMEM((CHUNK,), jnp.int32) for _ in range(4)],
            [pltpu.VMEM((2, CHUNK), jnp.int32) for _ in range(4)],
            [pltpu.VMEM((CHUNK, EMB), jnp.float32) for _ in range(2)],
            pltpu.VMEM((RPW * EMB,), jnp.float32),
            [pltpu.SemaphoreType.DMA for _ in range(4)],
            [pltpu.SemaphoreType.DMA for _ in range(4)],
            [pltpu.SemaphoreType.DMA for _ in range(2)],
        ],
        compiler_params=pltpu.CompilerParams(needs_layout_passes=False),
    )
    def k(x_ref, cols_ref, vr_ref, bnd_ref, out_ref,
          bnd_v, cols_v, vr_v, gath_v, acc_v, csem, vsem, gsem):
        wid = lax.axis_index("s") * 2 + lax.axis_index("c")
        row_base = wid * RPW
        iota16 = lax.iota(jnp.int32, 16)
        z16 = jnp.zeros((16,), jnp.int32)
        o16 = jnp.full((16,), 1, jnp.int32)
        rbv = jnp.full((16,), row_base, jnp.int32)

        pltpu.sync_copy(bnd_ref, bnd_v)
        widv = jnp.full((16,), wid, dtype=jnp.int32)
        e_lo = plsc.load_gather(bnd_v, [widv])[0]
        e_hi = plsc.load_gather(bnd_v, [widv + 1])[0]

        c_lo = e_lo // CHUNK
        c_hi = lax.div(e_hi + (CHUNK - 1), CHUNK)
        nc4 = lax.div(c_hi - c_lo + 3, 4)

        def clamp(c):
            return lax.min(c, n_chunks - 1)

        def stage(c, i):
            pltpu.async_copy(cols_ref.at[pl.ds(clamp(c) * CHUNK, CHUNK)],
                             cols_v[i], csem[i])
            pltpu.async_copy(vr_ref.at[clamp(c)], vr_v[i], vsem[i])

        def wait_cols(i):
            pltpu.make_async_copy(cols_ref.at[pl.ds(0, CHUNK)],
                                  cols_v[i], csem[i]).wait()

        def wait_vr(i):
            pltpu.make_async_copy(vr_ref.at[0], vr_v[i], vsem[i]).wait()

        def wait_gath(i):
            pltpu.make_async_copy(x_ref.at[pl.ds(0, CHUNK)],
                                  gath_v[i], gsem[i]).wait()

        stage(c_lo, 0)
        stage(c_lo + 1, 1)

        zeros = jnp.zeros((16,), jnp.float32)

        def zero_body(t, _):
            acc_v[pl.ds(t * 16, 16)] = zeros
            return 0

        lax.fori_loop(0, RPW * FEV, zero_body, 0)

        wait_cols(0)
        pltpu.async_copy(x_ref.at[cols_v[0]], gath_v[0], gsem[0])

        def compute(c, vr_i, gath_i):
            base = c * CHUNK
            k0 = lax.max(e_lo - base, 0)
            k1 = lax.min(e_hi - base, CHUNK)
            k0v = jnp.full((16,), k0, jnp.int32)
            k1v = jnp.full((16,), k1, jnp.int32)

            def group_body(g, _):
                basev = jnp.full((16,), g * 16, jnp.int32)
                for i in range(16):
                    ev = basev + i
                    vv = plsc.bitcast(plsc.load_gather(vr_i, [z16, ev]),
                                      jnp.float32)
                    rv = plsc.load_gather(vr_i, [o16, ev])
                    okv = jnp.logical_and(ev >= k0v, ev < k1v)
                    vvm = jnp.where(okv, vv, 0.0)
                    rr = jnp.where(okv, (rv - rbv) * EMB, 0)
                    gvs = [plsc.load_gather(gath_i, [ev, iota16 + (j * 16)])
                           for j in range(FEV)]
                    prods = [gv * vvm for gv in gvs]
                    for j in range(FEV):
                        plsc.addupdate_scatter(
                            acc_v, [rr + (iota16 + (j * 16))], prods[j])
                return 0

            lax.fori_loop(k0 // 16, lax.div(k1 + 15, 16), group_body, 0)

        def outer_body(it, _):
            c0 = c_lo + it * 4
            for b in range(4):
                c = c0 + b
                gb = b % 2
                stage(c + 2, (b + 2) % 4)
                wait_cols((b + 1) % 4)
                pltpu.async_copy(x_ref.at[cols_v[(b + 1) % 4]],
                                 gath_v[1 - gb], gsem[1 - gb])
                wait_gath(gb)
                wait_vr(b)
                compute(c, vr_v[b], gath_v[gb])
            return 0

        lax.fori_loop(0, nc4, outer_body, 0)

        wait_vr(0)
        wait_cols(1)
        wait_vr(1)
        wait_gath(0)

        pltpu.sync_copy(acc_v, out_ref.at[pl.ds(row_base * EMB, RPW * EMB)])

    out = k(x_hbm, cols, vr, bounds)
    return out.reshape(N_PAD, EMB)


def _mean_tc(x0, a1, a2, a3, b1, b2, b3):
    blk = N_PAD // 4

    def body(x0_r, a1_r, a2_r, a3_r, b1_r, b2_r, b3_r, o1_r, o2_r):
        x = x0_r[...]
        o1_r[...] = (x + a1_r[...] + a2_r[...] + a3_r[...]) * 0.25
        o2_r[...] = (x + b1_r[...] + b2_r[...] + b3_r[...]) * 0.25

    spec = pl.BlockSpec((blk, EMB), lambda i: (i, 0))
    o1, o2 = pl.pallas_call(
        body,
        grid=(4,),
        in_specs=[spec] * 7,
        out_specs=[spec, spec],
        out_shape=[jax.ShapeDtypeStruct((N_PAD, EMB), jnp.float32)] * 2,
    )(x0, a1, a2, a3, b1, b2, b3)
    return o1, o2


def _prep_mat(rc_v):
    rows, cols, vals = rc_v
    e = rows.shape[0]
    ep = ((e + CHUNK - 1) // CHUNK) * CHUNK
    pad = ep - e
    bounds = jnp.searchsorted(
        rows, jnp.arange(33, dtype=jnp.int32) * RPW).astype(jnp.int32)
    bounds = jnp.pad(bounds, (0, 15))
    rows_p = jnp.pad(rows.astype(jnp.int32), (0, pad),
                     constant_values=N_PAD - 1)
    cols_p = jnp.pad(cols.astype(jnp.int32), (0, pad))
    vals_p = jnp.pad(vals, (0, pad))
    nc = ep // CHUNK
    vr = jnp.stack([
        lax.bitcast_convert_type(vals_p, jnp.int32).reshape(nc, CHUNK),
        rows_p.reshape(nc, CHUNK),
    ], axis=1)
    return cols_p, vr, bounds


def kernel(ego_embeddings, sub1, sub2):
    x0 = jnp.pad(ego_embeddings, ((0, N_PAD - N_NODES), (0, 0)))

    def propagate(mats):
        outs = []
        x = x0
        for rc_v in mats:
            cols_p, vr, bounds = _prep_mat(rc_v)
            x = _spmm_sc(x, cols_p, vr, bounds)
            outs.append(x)
        return outs

    a1, a2, a3 = propagate(sub1)
    b1, b2, b3 = propagate(sub2)
    o1, o2 = _mean_tc(x0, a1, a2, a3, b1, b2, b3)
    o1 = o1[:N_NODES]
    o2 = o2[:N_NODES]
    half = N_NODES // 2
    return (o1[:half], o1[half:], o2[:half], o2[half:])
